# Initial kernel scaffold; baseline (speedup 1.0000x reference)
import jax
import jax.numpy as jnp
from jax.experimental import pallas as pl


def kernel(x):
    raise NotImplementedError("write your pallas kernel here")


# speedup vs baseline: 53.1885x; 53.1885x over previous
import jax
import jax.numpy as jnp
from jax.experimental import pallas as pl

TOPK = 8
SEQ = 8192
FEAT = 128
CHUNK = 32
_NEG = float("-inf")


def _kmax_body(x_ref, o_ref):
    def body(i, acc):
        v = x_ref[0, pl.ds(i * CHUNK, CHUNK), :]
        new = []
        for t in acc:
            hi = jnp.maximum(t, v)
            v = jnp.minimum(t, v)
            new.append(hi)
        return tuple(new)

    init = tuple(jnp.full((CHUNK, FEAT), _NEG, jnp.float32) for _ in range(TOPK))
    acc = jax.lax.fori_loop(0, SEQ // CHUNK, body, init)

    cand = jnp.concatenate(acc, axis=0)
    n = TOPK * CHUNK
    iota = jax.lax.broadcasted_iota(jnp.int32, (n, FEAT), 0)
    rows = []
    for _ in range(TOPK):
        m = jnp.max(cand, axis=0, keepdims=True)
        eq = cand == m
        first = jnp.min(jnp.where(eq, iota, n), axis=0, keepdims=True)
        cand = jnp.where(eq & (iota == first), _NEG, cand)
        rows.append(m)
    top = jnp.concatenate(rows, axis=0)
    o_ref[0] = top.T


def kernel(x):
    b, s, f = x.shape
    out = pl.pallas_call(
        _kmax_body,
        grid=(b,),
        in_specs=[pl.BlockSpec((1, s, f), lambda i: (i, 0, 0))],
        out_specs=pl.BlockSpec((1, f, TOPK), lambda i: (i, 0, 0)),
        out_shape=jax.ShapeDtypeStruct((b, f, TOPK), x.dtype),
    )(x)
    return out.reshape(b, f * TOPK)


# speedup vs baseline: 105.3818x; 1.9813x over previous
import jax
import jax.numpy as jnp
from jax.experimental import pallas as pl

TOPK = 8
SEQ = 8192
FEAT = 128
CHUNK = 8
_NEG = float("-inf")

_NET8 = (
    (0, 1), (2, 3), (4, 5), (6, 7),
    (0, 2), (1, 3), (4, 6), (5, 7),
    (1, 2), (5, 6), (0, 4), (3, 7),
    (1, 5), (2, 6),
    (1, 4), (3, 6),
    (2, 4), (3, 5),
    (3, 4),
)


def _kmax_body(x_ref, o_ref):
    group = 8 * CHUNK

    def body(i, acc):
        blk = x_ref[0, pl.ds(i * group, group), :]
        v = [blk[j * CHUNK:(j + 1) * CHUNK, :] for j in range(8)]
        for a, b in _NET8:
            hi = jnp.maximum(v[a], v[b])
            lo = jnp.minimum(v[a], v[b])
            v[a], v[b] = hi, lo
        h = [jnp.maximum(acc[j], v[7 - j]) for j in range(8)]
        for d in (4, 2, 1):
            nh = list(h)
            for s in range(0, 8, 2 * d):
                for t in range(s, s + d):
                    nh[t] = jnp.maximum(h[t], h[t + d])
                    nh[t + d] = jnp.minimum(h[t], h[t + d])
            h = nh
        return tuple(h)

    init = tuple(jnp.full((CHUNK, FEAT), _NEG, jnp.float32) for _ in range(TOPK))
    acc = jax.lax.fori_loop(0, SEQ // group, body, init)

    cand = jnp.concatenate(acc, axis=0)
    n = TOPK * CHUNK
    iota = jax.lax.broadcasted_iota(jnp.int32, (n, FEAT), 0)
    rows = []
    for _ in range(TOPK):
        m = jnp.max(cand, axis=0, keepdims=True)
        eq = cand == m
        first = jnp.min(jnp.where(eq, iota, n), axis=0, keepdims=True)
        cand = jnp.where(eq & (iota == first), _NEG, cand)
        rows.append(m)
    top = jnp.concatenate(rows, axis=0)
    o_ref[0] = top.T


def kernel(x):
    b, s, f = x.shape
    out = pl.pallas_call(
        _kmax_body,
        grid=(b,),
        in_specs=[pl.BlockSpec((1, s, f), lambda i: (i, 0, 0))],
        out_specs=pl.BlockSpec((1, f, TOPK), lambda i: (i, 0, 0)),
        out_shape=jax.ShapeDtypeStruct((b, f, TOPK), x.dtype),
    )(x)
    return out.reshape(b, f * TOPK)


# speedup vs baseline: 121.1048x; 1.1492x over previous
import jax
import jax.numpy as jnp
from jax.experimental import pallas as pl

TOPK = 8
SEQ = 8192
FEAT = 128
CHUNK = 16
_NEG = float("-inf")

_NET8 = (
    (0, 1), (2, 3), (4, 5), (6, 7),
    (0, 2), (1, 3), (4, 6), (5, 7),
    (1, 2), (5, 6), (0, 4), (3, 7),
    (1, 5), (2, 6),
    (1, 4), (3, 6),
    (2, 4), (3, 5),
    (3, 4),
)


def _kmax_body(x_ref, o_ref):
    group = 8 * CHUNK

    def body(i, acc):
        blk = x_ref[0, pl.ds(i * group, group), :]
        v = [blk[j * CHUNK:(j + 1) * CHUNK, :] for j in range(8)]
        for a, b in _NET8:
            hi = jnp.maximum(v[a], v[b])
            lo = jnp.minimum(v[a], v[b])
            v[a], v[b] = hi, lo
        h = [jnp.maximum(acc[j], v[7 - j]) for j in range(8)]
        for d in (4, 2, 1):
            nh = list(h)
            for s in range(0, 8, 2 * d):
                for t in range(s, s + d):
                    nh[t] = jnp.maximum(h[t], h[t + d])
                    nh[t + d] = jnp.minimum(h[t], h[t + d])
            h = nh
        return tuple(h)

    init = tuple(jnp.full((CHUNK, FEAT), _NEG, jnp.float32) for _ in range(TOPK))
    acc = jax.lax.fori_loop(0, SEQ // group, body, init)

    cand = jnp.concatenate(acc, axis=0)
    n = TOPK * CHUNK
    iota = jax.lax.broadcasted_iota(jnp.int32, (n, FEAT), 0)
    rows = []
    for _ in range(TOPK):
        m = jnp.max(cand, axis=0, keepdims=True)
        eq = cand == m
        first = jnp.min(jnp.where(eq, iota, n), axis=0, keepdims=True)
        cand = jnp.where(eq & (iota == first), _NEG, cand)
        rows.append(m)
    top = jnp.concatenate(rows, axis=0)
    o_ref[0] = top.T


def kernel(x):
    b, s, f = x.shape
    out = pl.pallas_call(
        _kmax_body,
        grid=(b,),
        in_specs=[pl.BlockSpec((1, s, f), lambda i: (i, 0, 0))],
        out_specs=pl.BlockSpec((1, f, TOPK), lambda i: (i, 0, 0)),
        out_shape=jax.ShapeDtypeStruct((b, f, TOPK), x.dtype),
    )(x)
    return out.reshape(b, f * TOPK)
